# baseline (speedup 1.0000x reference)
import functools

import jax
import jax.numpy as jnp
from jax import lax
from jax.experimental import pallas as pl
from jax.experimental.pallas import tpu as pltpu
from jax.experimental.pallas import tpu_sc as plsc

MAXLEN = 2048
NROWS = 16
NFLAT = 16384
LANES = 16
HALF = MAXLEN // 2

_mesh = plsc.VectorSubcoreMesh(
    core_axis_name="c", subcore_axis_name="s", num_cores=1, num_subcores=16
)


@functools.partial(
    pl.kernel,
    out_type=jax.ShapeDtypeStruct((NROWS, MAXLEN), jnp.float32),
    mesh=_mesh,
    scratch_types=[
        pltpu.VMEM((NFLAT,), jnp.float32),
        pltpu.VMEM((HALF,), jnp.float32),
        pltpu.VMEM((HALF,), jnp.float32),
        pltpu.VMEM((LANES,), jnp.int32),
        pltpu.SemaphoreType.DMA,
        pltpu.SemaphoreType.DMA,
        pltpu.SemaphoreType.DMA,
    ],
    compiler_params=pltpu.CompilerParams(needs_layout_passes=False),
)
def _encode_sc(
    flat_hbm, cu_hbm, out_hbm,
    flat_v, out0_v, out1_v, cu_v,
    sem_f, sem_o0, sem_o1,
):
    b = lax.axis_index("s")

    cf = pltpu.async_copy(flat_hbm, flat_v, sem_f)
    pltpu.sync_copy(cu_hbm.at[pl.ds(0, LANES)], cu_v)

    bvec = jnp.full((LANES,), b, dtype=jnp.int32)
    start = plsc.load_gather(cu_v, [bvec])
    end = plsc.load_gather(cu_v, [jnp.minimum(bvec + 1, NROWS - 1)])
    end = jnp.where(bvec == NROWS - 1, NFLAT, end)

    iota = lax.iota(jnp.int32, LANES)
    zero = jnp.zeros((LANES,), jnp.float32)
    sp = start + iota
    limv = start + jnp.minimum(end - start, MAXLEN)

    cf.wait()

    @plsc.parallel_loop(0, HALF, LANES, unroll=4)
    def _body0(j):
        t = sp + j
        vals = plsc.load_gather(flat_v, [jnp.minimum(t, NFLAT - 1)])
        out0_v[pl.ds(j, LANES)] = jnp.where(t < limv, vals, zero)

    o0 = pltpu.async_copy(out0_v, out_hbm.at[b, pl.ds(0, HALF)], sem_o0)

    @plsc.parallel_loop(HALF, MAXLEN, LANES, unroll=4)
    def _body1(j):
        t = sp + j
        vals = plsc.load_gather(flat_v, [jnp.minimum(t, NFLAT - 1)])
        out1_v[pl.ds(j - HALF, LANES)] = jnp.where(t < limv, vals, zero)

    o1 = pltpu.async_copy(out1_v, out_hbm.at[b, pl.ds(HALF, HALF)], sem_o1)

    o0.wait()
    o1.wait()


def kernel(flat, cu_seqlens):
    return _encode_sc(flat, cu_seqlens.astype(jnp.int32))


# speedup vs baseline: 1.0513x; 1.0513x over previous
import functools

import jax
import jax.numpy as jnp
from jax import lax
from jax.experimental import pallas as pl
from jax.experimental.pallas import tpu as pltpu
from jax.experimental.pallas import tpu_sc as plsc

MAXLEN = 2048
NROWS = 16
NFLAT = 16384
LANES = 16
HALF = MAXLEN // 2
WIN = HALF + 32

_mesh = plsc.VectorSubcoreMesh(
    core_axis_name="c", subcore_axis_name="s", num_cores=1, num_subcores=16
)


@functools.partial(
    pl.kernel,
    out_type=jax.ShapeDtypeStruct((NROWS, MAXLEN), jnp.float32),
    mesh=_mesh,
    scratch_types=[
        pltpu.VMEM((WIN,), jnp.float32),
        pltpu.VMEM((WIN,), jnp.float32),
        pltpu.VMEM((HALF,), jnp.float32),
        pltpu.VMEM((HALF,), jnp.float32),
        pltpu.VMEM((LANES,), jnp.int32),
        pltpu.SemaphoreType.DMA,
        pltpu.SemaphoreType.DMA,
        pltpu.SemaphoreType.DMA,
        pltpu.SemaphoreType.DMA,
    ],
    compiler_params=pltpu.CompilerParams(needs_layout_passes=False),
)
def _encode_sc(
    flat_hbm, cu_hbm, out_hbm,
    win0_v, win1_v, out0_v, out1_v, cu_v,
    sem_w0, sem_w1, sem_o0, sem_o1,
):
    b = lax.axis_index("s")

    pltpu.sync_copy(cu_hbm.at[pl.ds(0, LANES)], cu_v)

    bvec = jnp.full((LANES,), b, dtype=jnp.int32)
    start = plsc.load_gather(cu_v, [bvec])
    end = plsc.load_gather(cu_v, [jnp.minimum(bvec + 1, NROWS - 1)])
    end = jnp.where(bvec == NROWS - 1, NFLAT, end)

    s0 = jnp.max(start, axis=0)
    a0 = jnp.clip(s0 & -16, 0, NFLAT - WIN)
    a0 = pl.multiple_of(a0, 16)
    a1 = jnp.clip((s0 + HALF) & -16, 0, NFLAT - WIN)
    a1 = pl.multiple_of(a1, 16)
    c0 = pltpu.async_copy(flat_hbm.at[pl.ds(a0, WIN)], win0_v, sem_w0)
    c1 = pltpu.async_copy(flat_hbm.at[pl.ds(a1, WIN)], win1_v, sem_w1)

    iota = lax.iota(jnp.int32, LANES)
    zero = jnp.zeros((LANES,), jnp.float32)
    sp = start + iota
    limv = start + jnp.minimum(end - start, MAXLEN)
    a0v = jnp.full((LANES,), a0, dtype=jnp.int32)
    a1v = jnp.full((LANES,), a1, dtype=jnp.int32)

    c0.wait()

    @plsc.parallel_loop(0, HALF, LANES, unroll=4)
    def _body0(j):
        t = sp + j
        rel = jnp.minimum(t - a0v, WIN - 1)
        vals = plsc.load_gather(win0_v, [rel])
        out0_v[pl.ds(j, LANES)] = jnp.where(t < limv, vals, zero)

    o0 = pltpu.async_copy(out0_v, out_hbm.at[b, pl.ds(0, HALF)], sem_o0)

    c1.wait()

    @plsc.parallel_loop(HALF, MAXLEN, LANES, unroll=4)
    def _body1(j):
        t = sp + j
        rel = jnp.minimum(t - a1v, WIN - 1)
        vals = plsc.load_gather(win1_v, [rel])
        out1_v[pl.ds(j - HALF, LANES)] = jnp.where(t < limv, vals, zero)

    o1 = pltpu.async_copy(out1_v, out_hbm.at[b, pl.ds(HALF, HALF)], sem_o1)

    o0.wait()
    o1.wait()


def kernel(flat, cu_seqlens):
    return _encode_sc(flat, cu_seqlens.astype(jnp.int32))
